# baseline (speedup 1.0000x reference)
import functools

import jax
import jax.numpy as jnp
from jax import lax
from jax.experimental import pallas as pl
from jax.experimental.pallas import tpu as pltpu
from jax.experimental.pallas import tpu_sc as plsc

_F32 = jnp.float32
_PREC = lax.Precision.HIGHEST


def _bf16_dot(a, b):
    return jnp.dot(a.astype(jnp.bfloat16), b.astype(jnp.bfloat16),
                   preferred_element_type=_F32)


def _main_body(tok_ref, tf_ref, wccm_ref, bccm_ref, win_ref, bin_ref,
               cb_ref, cbt_ref, idx_ref, loss_ref):
    b = pl.program_id(0)
    nb, l, td = tf_ref.shape
    d, k = cbt_ref.shape
    r = nb * l

    img_bf = tok_ref[...].astype(jnp.bfloat16)
    wccm_bf = wccm_ref[...].astype(jnp.bfloat16)
    conds = []
    for i in range(nb):
        c_i = lax.dot_general(img_bf[i], wccm_bf, (((0,), (0,)), ((), ())),
                              preferred_element_type=_F32)
        c_i = c_i + bccm_ref[...]
        conds.append(jnp.mean(c_i, axis=0, keepdims=True))
    cond = jnp.concatenate(conds, axis=0)

    h = (tf_ref[...] + cond[:, None, :]).reshape(r, td)
    z = _bf16_dot(h, win_ref[...]) + bin_ref[...]
    scores = _bf16_dot(z, cbt_ref[...])
    cb2 = jnp.sum(cb_ref[...] * cb_ref[...], axis=1).reshape(1, k)
    z2 = jnp.sum(z * z, axis=1, keepdims=True)
    dist = z2 + cb2 - 2.0 * scores
    minval = jnp.min(dist, axis=1, keepdims=True)
    iota = lax.broadcasted_iota(jnp.int32, (r, k), 1)
    idx = jnp.min(jnp.where(dist == minval, iota, jnp.int32(k)),
                  axis=1, keepdims=True)
    idx_ref[...] = idx

    contrib = jnp.sum(minval, axis=0, keepdims=True)

    @pl.when(b == 0)
    def _():
        loss_ref[...] = jnp.zeros_like(loss_ref)

    loss_ref[...] += contrib


def _cw_body(cb_ref, wout_ref, bout_ref, cw_ref):
    cw_ref[...] = _bf16_dot(cb_ref[...], wout_ref[...]) + bout_ref[...]


def _make_sc_gather(n_rows, td):
    info = plsc.get_sparse_core_info()
    nc, ns = info.num_cores, info.num_subcores
    nw = nc * ns
    rows_per_w = n_rows // nw
    mesh = plsc.VectorSubcoreMesh(core_axis_name="c", subcore_axis_name="s")

    @functools.partial(
        pl.kernel, mesh=mesh,
        out_type=jax.ShapeDtypeStruct((n_rows, td), _F32),
        scratch_types=[
            pltpu.VMEM((rows_per_w,), jnp.int32),
            pltpu.VMEM((rows_per_w, td), _F32),
            pltpu.SemaphoreType.DMA,
        ],
    )
    def sc_gather(cw_hbm, idx_hbm, out_hbm, idx_v, rows_v, sem):
        wid = lax.axis_index("s") * nc + lax.axis_index("c")
        base = wid * rows_per_w
        pltpu.sync_copy(idx_hbm.at[pl.ds(base, rows_per_w)], idx_v)
        pltpu.async_copy(cw_hbm.at[idx_v], rows_v, sem).wait()
        pltpu.sync_copy(rows_v, out_hbm.at[pl.ds(base, rows_per_w)])

    return sc_gather


def kernel(text_features, text_attention_mask, img_features, W_ccm, b_ccm,
           W_in, b_in, codebook, W_out, b_out):
    B, L, TD = text_features.shape
    _, C, H, W = img_features.shape
    K, D = codebook.shape
    HW = H * W

    img3 = img_features.reshape(B, C, HW)
    cb_t = codebook.T
    bccm2 = b_ccm.reshape(1, TD)
    bin2 = b_in.reshape(1, D)
    bout2 = b_out.reshape(1, TD)

    NB = 2
    grid = B // NB
    R = NB * L

    idx, loss_sum = pl.pallas_call(
        _main_body,
        grid=(grid,),
        in_specs=[
            pl.BlockSpec((NB, C, HW), lambda b: (b, 0, 0)),
            pl.BlockSpec((NB, L, TD), lambda b: (b, 0, 0)),
            pl.BlockSpec((C, TD), lambda b: (0, 0)),
            pl.BlockSpec((1, TD), lambda b: (0, 0)),
            pl.BlockSpec((TD, D), lambda b: (0, 0)),
            pl.BlockSpec((1, D), lambda b: (0, 0)),
            pl.BlockSpec((K, D), lambda b: (0, 0)),
            pl.BlockSpec((D, K), lambda b: (0, 0)),
        ],
        out_specs=[
            pl.BlockSpec((R, 1), lambda b: (b, 0)),
            pl.BlockSpec((1, 1), lambda b: (0, 0)),
        ],
        out_shape=[
            jax.ShapeDtypeStruct((B * L, 1), jnp.int32),
            jax.ShapeDtypeStruct((1, 1), _F32),
        ],
    )(img3, text_features, W_ccm, bccm2, W_in, bin2, codebook, cb_t)

    cw = pl.pallas_call(
        _cw_body,
        out_shape=jax.ShapeDtypeStruct((K, TD), _F32),
    )(codebook, W_out, bout2)

    gathered = _make_sc_gather(B * L, TD)(cw, idx.reshape(B * L))
    out = gathered.reshape(B, L, TD)

    vq_loss = (loss_sum[0, 0] * (1.25 / (B * L * D))).astype(_F32)
    ccm_loss = jnp.zeros((), dtype=_F32)
    return out, text_attention_mask, ccm_loss, vq_loss


# speedup vs baseline: 1.2999x; 1.2999x over previous
import jax
import jax.numpy as jnp
from jax import lax
from jax.experimental import pallas as pl
from jax.experimental.pallas import tpu as pltpu

_F32 = jnp.float32


def _bf16_dot(a, b):
    return jnp.dot(a.astype(jnp.bfloat16), b.astype(jnp.bfloat16),
                   preferred_element_type=_F32)


def _main_body(tok_ref, tf_ref, wccm_ref, bccm_ref, win_ref, bin_ref,
               cb_ref, cbt_ref, wout_ref, bout_ref,
               out_ref, loss_ref, cw_ref):
    b = pl.program_id(0)
    nb, l, td = tf_ref.shape
    d, k = cbt_ref.shape
    r = nb * l

    @pl.when(b == 0)
    def _():
        cw_ref[...] = (_bf16_dot(cb_ref[...], wout_ref[...])
                       + bout_ref[...]).astype(jnp.bfloat16)

    img_bf = tok_ref[...].astype(jnp.bfloat16)
    wccm_bf = wccm_ref[...].astype(jnp.bfloat16)
    conds = []
    for i in range(nb):
        c_i = lax.dot_general(img_bf[i], wccm_bf, (((0,), (0,)), ((), ())),
                              preferred_element_type=_F32)
        c_i = c_i + bccm_ref[...]
        conds.append(jnp.mean(c_i, axis=0, keepdims=True))
    cond = jnp.concatenate(conds, axis=0)

    h = (tf_ref[...] + cond[:, None, :]).reshape(r, td)
    z = _bf16_dot(h, win_ref[...]) + bin_ref[...]
    scores = _bf16_dot(z, cbt_ref[...])
    cb2 = jnp.sum(cb_ref[...] * cb_ref[...], axis=1).reshape(1, k)
    z2 = jnp.sum(z * z, axis=1, keepdims=True)
    dist = z2 + cb2 - 2.0 * scores
    minval = jnp.min(dist, axis=1, keepdims=True)
    iota = lax.broadcasted_iota(jnp.int32, (r, k), 1)
    idx = jnp.min(jnp.where(dist == minval, iota, jnp.int32(k)),
                  axis=1, keepdims=True)

    onehot = (iota == idx).astype(jnp.bfloat16)
    out_ref[...] = jnp.dot(onehot, cw_ref[...],
                           preferred_element_type=_F32)

    contrib = jnp.sum(minval, axis=0, keepdims=True)

    @pl.when(b == 0)
    def _():
        loss_ref[...] = jnp.zeros_like(loss_ref)

    loss_ref[...] += contrib


def kernel(text_features, text_attention_mask, img_features, W_ccm, b_ccm,
           W_in, b_in, codebook, W_out, b_out):
    B, L, TD = text_features.shape
    _, C, H, W = img_features.shape
    K, D = codebook.shape
    HW = H * W

    img3 = img_features.reshape(B, C, HW)
    cb_t = codebook.T
    bccm2 = b_ccm.reshape(1, TD)
    bin2 = b_in.reshape(1, D)
    bout2 = b_out.reshape(1, TD)

    NB = 2
    grid = B // NB
    R = NB * L

    out2, loss_sum = pl.pallas_call(
        _main_body,
        grid=(grid,),
        in_specs=[
            pl.BlockSpec((NB, C, HW), lambda b: (b, 0, 0)),
            pl.BlockSpec((NB, L, TD), lambda b: (b, 0, 0)),
            pl.BlockSpec((C, TD), lambda b: (0, 0)),
            pl.BlockSpec((1, TD), lambda b: (0, 0)),
            pl.BlockSpec((TD, D), lambda b: (0, 0)),
            pl.BlockSpec((1, D), lambda b: (0, 0)),
            pl.BlockSpec((K, D), lambda b: (0, 0)),
            pl.BlockSpec((D, K), lambda b: (0, 0)),
            pl.BlockSpec((D, TD), lambda b: (0, 0)),
            pl.BlockSpec((1, TD), lambda b: (0, 0)),
        ],
        out_specs=[
            pl.BlockSpec((R, TD), lambda b: (b, 0)),
            pl.BlockSpec((1, 1), lambda b: (0, 0)),
        ],
        out_shape=[
            jax.ShapeDtypeStruct((B * L, TD), _F32),
            jax.ShapeDtypeStruct((1, 1), _F32),
        ],
        scratch_shapes=[pltpu.VMEM((K, TD), jnp.bfloat16)],
    )(img3, text_features, W_ccm, bccm2, W_in, bin2, codebook, cb_t,
      W_out, bout2)

    out = out2.reshape(B, L, TD)
    vq_loss = (loss_sum[0, 0] * (1.25 / (B * L * D))).astype(_F32)
    ccm_loss = jnp.zeros((), dtype=_F32)
    return out, text_attention_mask, ccm_loss, vq_loss


# speedup vs baseline: 1.4170x; 1.0901x over previous
import jax
import jax.numpy as jnp
from jax import lax
from jax.experimental import pallas as pl
from jax.experimental.pallas import tpu as pltpu

_F32 = jnp.float32


def _bf16_dot(a, b):
    return jnp.dot(a.astype(jnp.bfloat16), b.astype(jnp.bfloat16),
                   preferred_element_type=_F32)


def _main_body(tok_ref, tf_ref, wccm_ref, bccm_ref, win_ref, bin_ref,
               cb_ref, cbt_ref, wout_ref, bout_ref,
               out_ref, loss_ref, cw_ref):
    b = pl.program_id(0)
    nb, l, td = tf_ref.shape
    d, k = cbt_ref.shape
    r = nb * l

    @pl.when(b == 0)
    def _():
        cw_ref[...] = (_bf16_dot(cb_ref[...], wout_ref[...])
                       + bout_ref[...]).astype(jnp.bfloat16)

    img_bf = tok_ref[...].astype(jnp.bfloat16)
    wccm_bf = wccm_ref[...].astype(jnp.bfloat16)
    conds = []
    for i in range(nb):
        c_i = lax.dot_general(img_bf[i], wccm_bf, (((0,), (0,)), ((), ())),
                              preferred_element_type=_F32)
        c_i = c_i + bccm_ref[...]
        conds.append(jnp.mean(c_i, axis=0, keepdims=True))
    cond = jnp.concatenate(conds, axis=0)

    h = (tf_ref[...] + cond[:, None, :]).reshape(r, td)
    z = _bf16_dot(h, win_ref[...]) + bin_ref[...]
    scores = _bf16_dot(z, cbt_ref[...])
    cb2 = jnp.sum(cb_ref[...] * cb_ref[...], axis=1).reshape(1, k)
    z2 = jnp.sum(z * z, axis=1, keepdims=True)
    dist = z2 + cb2 - 2.0 * scores
    minval = jnp.min(dist, axis=1, keepdims=True)
    iota = lax.broadcasted_iota(jnp.int32, (r, k), 1)
    idx = jnp.min(jnp.where(dist == minval, iota, jnp.int32(k)),
                  axis=1, keepdims=True)

    onehot = (iota == idx).astype(jnp.bfloat16)
    out_ref[...] = jnp.dot(onehot, cw_ref[...],
                           preferred_element_type=_F32)

    contrib = jnp.sum(minval, axis=0, keepdims=True)

    @pl.when(b == 0)
    def _():
        loss_ref[...] = jnp.zeros_like(loss_ref)

    loss_ref[...] += contrib


def kernel(text_features, text_attention_mask, img_features, W_ccm, b_ccm,
           W_in, b_in, codebook, W_out, b_out):
    B, L, TD = text_features.shape
    _, C, H, W = img_features.shape
    K, D = codebook.shape
    HW = H * W

    img3 = img_features.reshape(B, C, HW)
    cb_t = codebook.T
    bccm2 = b_ccm.reshape(1, TD)
    bin2 = b_in.reshape(1, D)
    bout2 = b_out.reshape(1, TD)

    NB = 4
    grid = B // NB
    R = NB * L

    out2, loss_sum = pl.pallas_call(
        _main_body,
        grid=(grid,),
        in_specs=[
            pl.BlockSpec((NB, C, HW), lambda b: (b, 0, 0)),
            pl.BlockSpec((NB, L, TD), lambda b: (b, 0, 0)),
            pl.BlockSpec((C, TD), lambda b: (0, 0)),
            pl.BlockSpec((1, TD), lambda b: (0, 0)),
            pl.BlockSpec((TD, D), lambda b: (0, 0)),
            pl.BlockSpec((1, D), lambda b: (0, 0)),
            pl.BlockSpec((K, D), lambda b: (0, 0)),
            pl.BlockSpec((D, K), lambda b: (0, 0)),
            pl.BlockSpec((D, TD), lambda b: (0, 0)),
            pl.BlockSpec((1, TD), lambda b: (0, 0)),
        ],
        out_specs=[
            pl.BlockSpec((R, TD), lambda b: (b, 0)),
            pl.BlockSpec((1, 1), lambda b: (0, 0)),
        ],
        out_shape=[
            jax.ShapeDtypeStruct((B * L, TD), _F32),
            jax.ShapeDtypeStruct((1, 1), _F32),
        ],
        scratch_shapes=[pltpu.VMEM((K, TD), jnp.bfloat16)],
    )(img3, text_features, W_ccm, bccm2, W_in, bin2, codebook, cb_t,
      W_out, bout2)

    out = out2.reshape(B, L, TD)
    vq_loss = (loss_sum[0, 0] * (1.25 / (B * L * D))).astype(_F32)
    ccm_loss = jnp.zeros((), dtype=_F32)
    return out, text_attention_mask, ccm_loss, vq_loss


# speedup vs baseline: 1.4374x; 1.0144x over previous
import jax
import jax.numpy as jnp
from jax import lax
from jax.experimental import pallas as pl
from jax.experimental.pallas import tpu as pltpu

_F32 = jnp.float32


def _bf16_dot(a, b):
    return jnp.dot(a.astype(jnp.bfloat16), b.astype(jnp.bfloat16),
                   preferred_element_type=_F32)


def _main_body(tok_ref, tf_ref, wccm_ref, bccm_ref, win_ref, bin_ref,
               cb_ref, cbt_ref, wout_ref, bout_ref,
               out_ref, loss_ref, cw_ref):
    b = pl.program_id(0)
    nb, l, td = tf_ref.shape
    d, k = cbt_ref.shape
    r = nb * l

    @pl.when(b == 0)
    def _():
        cw_ref[...] = (_bf16_dot(cb_ref[...], wout_ref[...])
                       + bout_ref[...]).astype(jnp.bfloat16)

    img_bf = tok_ref[...].astype(jnp.bfloat16)
    wccm_bf = wccm_ref[...].astype(jnp.bfloat16)
    conds = []
    for i in range(nb):
        c_i = lax.dot_general(img_bf[i], wccm_bf, (((0,), (0,)), ((), ())),
                              preferred_element_type=_F32)
        c_i = c_i + bccm_ref[...]
        conds.append(jnp.mean(c_i, axis=0, keepdims=True))
    cond = jnp.concatenate(conds, axis=0)

    h = (tf_ref[...] + cond[:, None, :]).reshape(r, td)
    z = _bf16_dot(h, win_ref[...]) + bin_ref[...]
    scores = _bf16_dot(z, cbt_ref[...])
    cb2 = jnp.sum(cb_ref[...] * cb_ref[...], axis=1).reshape(1, k)
    z2 = jnp.sum(z * z, axis=1, keepdims=True)
    dist = z2 + cb2 - 2.0 * scores
    minval = jnp.min(dist, axis=1, keepdims=True)
    iota = lax.broadcasted_iota(jnp.int32, (r, k), 1)
    idx = jnp.min(jnp.where(dist == minval, iota, jnp.int32(k)),
                  axis=1, keepdims=True)

    onehot = (iota == idx).astype(jnp.bfloat16)
    out_ref[...] = jnp.dot(onehot, cw_ref[...],
                           preferred_element_type=_F32)

    contrib = jnp.sum(minval, axis=0, keepdims=True)

    @pl.when(b == 0)
    def _():
        loss_ref[...] = jnp.zeros_like(loss_ref)

    loss_ref[...] += contrib


def kernel(text_features, text_attention_mask, img_features, W_ccm, b_ccm,
           W_in, b_in, codebook, W_out, b_out):
    B, L, TD = text_features.shape
    _, C, H, W = img_features.shape
    K, D = codebook.shape
    HW = H * W

    img3 = img_features.reshape(B, C, HW)
    cb_t = codebook.T
    bccm2 = b_ccm.reshape(1, TD)
    bin2 = b_in.reshape(1, D)
    bout2 = b_out.reshape(1, TD)

    NB = 8
    grid = B // NB
    R = NB * L

    out2, loss_sum = pl.pallas_call(
        _main_body,
        grid=(grid,),
        in_specs=[
            pl.BlockSpec((NB, C, HW), lambda b: (b, 0, 0)),
            pl.BlockSpec((NB, L, TD), lambda b: (b, 0, 0)),
            pl.BlockSpec((C, TD), lambda b: (0, 0)),
            pl.BlockSpec((1, TD), lambda b: (0, 0)),
            pl.BlockSpec((TD, D), lambda b: (0, 0)),
            pl.BlockSpec((1, D), lambda b: (0, 0)),
            pl.BlockSpec((K, D), lambda b: (0, 0)),
            pl.BlockSpec((D, K), lambda b: (0, 0)),
            pl.BlockSpec((D, TD), lambda b: (0, 0)),
            pl.BlockSpec((1, TD), lambda b: (0, 0)),
        ],
        out_specs=[
            pl.BlockSpec((R, TD), lambda b: (b, 0)),
            pl.BlockSpec((1, 1), lambda b: (0, 0)),
        ],
        out_shape=[
            jax.ShapeDtypeStruct((B * L, TD), _F32),
            jax.ShapeDtypeStruct((1, 1), _F32),
        ],
        scratch_shapes=[pltpu.VMEM((K, TD), jnp.bfloat16)],
    )(img3, text_features, W_ccm, bccm2, W_in, bin2, codebook, cb_t,
      W_out, bout2)

    out = out2.reshape(B, L, TD)
    vq_loss = (loss_sum[0, 0] * (1.25 / (B * L * D))).astype(_F32)
    ccm_loss = jnp.zeros((), dtype=_F32)
    return out, text_attention_mask, ccm_loss, vq_loss


# speedup vs baseline: 1.5087x; 1.0496x over previous
import jax
import jax.numpy as jnp
from jax import lax
from jax.experimental import pallas as pl
from jax.experimental.pallas import tpu as pltpu

_F32 = jnp.float32


def _bf16_dot(a, b):
    return jnp.dot(a.astype(jnp.bfloat16), b.astype(jnp.bfloat16),
                   preferred_element_type=_F32)


def _main_body(tok_ref, tf_ref, wccm_ref, bccm_ref, win_ref, bin_ref,
               cb_ref, wout_ref, bout_ref,
               out_ref, loss_ref, cw_ref):
    b = pl.program_id(0)
    nb, l, td = tf_ref.shape
    k, d = cb_ref.shape
    r = nb * l

    @pl.when(b == 0)
    def _():
        cw_ref[...] = (_bf16_dot(cb_ref[...], wout_ref[...])
                       + bout_ref[...]).astype(jnp.bfloat16)

    wccm_bf = wccm_ref[...].astype(jnp.bfloat16)
    conds = []
    for i in range(nb):
        c_i = jnp.dot(tok_ref[:, i, :].astype(jnp.bfloat16), wccm_bf,
                      preferred_element_type=_F32)
        c_i = c_i + bccm_ref[...]
        conds.append(jnp.mean(c_i, axis=0, keepdims=True))
    cond = jnp.concatenate(conds, axis=0)

    h = (tf_ref[...] + cond[:, None, :]).reshape(r, td)
    z = _bf16_dot(h, win_ref[...]) + bin_ref[...]
    cb_bf = cb_ref[...].astype(jnp.bfloat16)
    scores = lax.dot_general(z.astype(jnp.bfloat16), cb_bf,
                             (((1,), (1,)), ((), ())),
                             preferred_element_type=_F32)
    cb2 = jnp.sum(cb_ref[...] * cb_ref[...], axis=1).reshape(1, k)
    z2 = jnp.sum(z * z, axis=1, keepdims=True)
    dist = z2 + cb2 - 2.0 * scores
    minval = jnp.min(dist, axis=1, keepdims=True)
    iota = lax.broadcasted_iota(jnp.int32, (r, k), 1)
    idx = jnp.min(jnp.where(dist == minval, iota, jnp.int32(k)),
                  axis=1, keepdims=True)

    onehot = (iota == idx).astype(jnp.bfloat16)
    out_ref[...] = jnp.dot(onehot, cw_ref[...],
                           preferred_element_type=_F32)

    contrib = jnp.sum(minval, axis=0, keepdims=True)

    @pl.when(b == 0)
    def _():
        loss_ref[...] = jnp.zeros_like(loss_ref)

    loss_ref[...] += contrib


def kernel(text_features, text_attention_mask, img_features, W_ccm, b_ccm,
           W_in, b_in, codebook, W_out, b_out):
    B, L, TD = text_features.shape
    _, C, H, W = img_features.shape
    K, D = codebook.shape
    HW = H * W

    img_tok = jnp.transpose(img_features.reshape(B, C, HW), (2, 0, 1))
    bccm2 = b_ccm.reshape(1, TD)
    bin2 = b_in.reshape(1, D)
    bout2 = b_out.reshape(1, TD)

    NB = 8
    grid = B // NB
    R = NB * L

    out2, loss_sum = pl.pallas_call(
        _main_body,
        grid=(grid,),
        in_specs=[
            pl.BlockSpec((HW, NB, C), lambda b: (0, b, 0)),
            pl.BlockSpec((NB, L, TD), lambda b: (b, 0, 0)),
            pl.BlockSpec((C, TD), lambda b: (0, 0)),
            pl.BlockSpec((1, TD), lambda b: (0, 0)),
            pl.BlockSpec((TD, D), lambda b: (0, 0)),
            pl.BlockSpec((1, D), lambda b: (0, 0)),
            pl.BlockSpec((K, D), lambda b: (0, 0)),
            pl.BlockSpec((D, TD), lambda b: (0, 0)),
            pl.BlockSpec((1, TD), lambda b: (0, 0)),
        ],
        out_specs=[
            pl.BlockSpec((R, TD), lambda b: (b, 0)),
            pl.BlockSpec((1, 1), lambda b: (0, 0)),
        ],
        out_shape=[
            jax.ShapeDtypeStruct((B * L, TD), _F32),
            jax.ShapeDtypeStruct((1, 1), _F32),
        ],
        scratch_shapes=[pltpu.VMEM((K, TD), jnp.bfloat16)],
    )(img_tok, text_features, W_ccm, bccm2, W_in, bin2, codebook,
      W_out, bout2)

    out = out2.reshape(B, L, TD)
    vq_loss = (loss_sum[0, 0] * (1.25 / (B * L * D))).astype(_F32)
    ccm_loss = jnp.zeros((), dtype=_F32)
    return out, text_attention_mask, ccm_loss, vq_loss


# speedup vs baseline: 2.2533x; 1.4936x over previous
import jax
import jax.numpy as jnp
from jax import lax
from jax.experimental import pallas as pl
from jax.experimental.pallas import tpu as pltpu

_F32 = jnp.float32


def _bf16_dot(a, b):
    return jnp.dot(a.astype(jnp.bfloat16), b.astype(jnp.bfloat16),
                   preferred_element_type=_F32)


def _main_body(tok_ref, tf_ref, wccm_ref, bccm_ref, win_ref, bin_ref,
               cb_ref, wout_ref, bout_ref,
               out_ref, loss_ref, cw_ref):
    b = pl.program_id(0)
    nb, l, td = tf_ref.shape
    k, d = cb_ref.shape
    r = nb * l

    @pl.when(b == 0)
    def _():
        cw_ref[...] = (_bf16_dot(cb_ref[...], wout_ref[...])
                       + bout_ref[...]).astype(jnp.bfloat16)

    hw = tok_ref.shape[0]
    wccm_bf = wccm_ref[...].astype(jnp.bfloat16)
    tok = tok_ref[...].astype(jnp.bfloat16).reshape(hw * nb, td)
    c = jnp.dot(tok, wccm_bf, preferred_element_type=_F32)
    c = c + bccm_ref[...]
    cond = jnp.mean(c.reshape(hw, nb, td), axis=0)

    h = (tf_ref[...] + cond[:, None, :]).reshape(r, td)
    z = _bf16_dot(h, win_ref[...]) + bin_ref[...]
    cb_bf = cb_ref[...].astype(jnp.bfloat16)
    scores = lax.dot_general(z.astype(jnp.bfloat16), cb_bf,
                             (((1,), (1,)), ((), ())),
                             preferred_element_type=_F32)
    cb2 = jnp.sum(cb_ref[...] * cb_ref[...], axis=1).reshape(1, k)
    z2 = jnp.sum(z * z, axis=1, keepdims=True)
    dist = z2 + cb2 - 2.0 * scores
    minval = jnp.min(dist, axis=1, keepdims=True)
    iota = lax.broadcasted_iota(jnp.int32, (r, k), 1)
    idx = jnp.min(jnp.where(dist == minval, iota, jnp.int32(k)),
                  axis=1, keepdims=True)

    onehot = (iota == idx).astype(jnp.bfloat16)
    out_ref[...] = jnp.dot(onehot, cw_ref[...],
                           preferred_element_type=_F32)

    contrib = jnp.sum(minval, axis=0, keepdims=True)

    @pl.when(b == 0)
    def _():
        loss_ref[...] = jnp.zeros_like(loss_ref)

    loss_ref[...] += contrib


def kernel(text_features, text_attention_mask, img_features, W_ccm, b_ccm,
           W_in, b_in, codebook, W_out, b_out):
    B, L, TD = text_features.shape
    _, C, H, W = img_features.shape
    K, D = codebook.shape
    HW = H * W

    img_tok = jnp.transpose(img_features.reshape(B, C, HW), (2, 0, 1))
    bccm2 = b_ccm.reshape(1, TD)
    bin2 = b_in.reshape(1, D)
    bout2 = b_out.reshape(1, TD)

    NB = 8
    grid = B // NB
    R = NB * L

    out2, loss_sum = pl.pallas_call(
        _main_body,
        grid=(grid,),
        in_specs=[
            pl.BlockSpec((HW, NB, C), lambda b: (0, b, 0)),
            pl.BlockSpec((NB, L, TD), lambda b: (b, 0, 0)),
            pl.BlockSpec((C, TD), lambda b: (0, 0)),
            pl.BlockSpec((1, TD), lambda b: (0, 0)),
            pl.BlockSpec((TD, D), lambda b: (0, 0)),
            pl.BlockSpec((1, D), lambda b: (0, 0)),
            pl.BlockSpec((K, D), lambda b: (0, 0)),
            pl.BlockSpec((D, TD), lambda b: (0, 0)),
            pl.BlockSpec((1, TD), lambda b: (0, 0)),
        ],
        out_specs=[
            pl.BlockSpec((R, TD), lambda b: (b, 0)),
            pl.BlockSpec((1, 1), lambda b: (0, 0)),
        ],
        out_shape=[
            jax.ShapeDtypeStruct((B * L, TD), _F32),
            jax.ShapeDtypeStruct((1, 1), _F32),
        ],
        scratch_shapes=[pltpu.VMEM((K, TD), jnp.bfloat16)],
    )(img_tok, text_features, W_ccm, bccm2, W_in, bin2, codebook,
      W_out, bout2)

    out = out2.reshape(B, L, TD)
    vq_loss = (loss_sum[0, 0] * (1.25 / (B * L * D))).astype(_F32)
    ccm_loss = jnp.zeros((), dtype=_F32)
    return out, text_attention_mask, ccm_loss, vq_loss
